# Initial kernel scaffold; baseline (speedup 1.0000x reference)
import jax
import jax.numpy as jnp
from jax.experimental import pallas as pl


def kernel(x):
    raise NotImplementedError("write your pallas kernel here")


# speedup vs baseline: 1.1472x; 1.1472x over previous
import functools

import jax
import jax.numpy as jnp
from jax import lax
from jax.experimental import pallas as pl
from jax.experimental.pallas import tpu as pltpu
from jax.experimental.pallas import tpu_sc as plsc

B = 16
N = 50000
K = 2000
F = 64
L = 16
NV = N // L
KL = K // L
CAND = K + 48
GCH = 200
NCH = K // GCH
NBIN = 256


def _s0(v):
    return lax.squeeze(lax.slice(v, (0,), (1,)), (0,))


def _keys_tc(x):
    nb = 400
    nj = N // nb

    def body(x_ref, d_ref):
        k = x_ref[:, 0, :, F - 1]
        u = lax.bitcast_convert_type(k, jnp.uint32)
        u = jnp.where(u == jnp.uint32(0x80000000), jnp.uint32(0), u)
        neg = u >= jnp.uint32(0x80000000)
        d = jnp.where(neg, u, u ^ jnp.uint32(0x7FFFFFFF))
        d_ref[0, :, :] = lax.bitcast_convert_type(d, jnp.int32)

    dd_t = pl.pallas_call(
        body,
        grid=(nj,),
        in_specs=[pl.BlockSpec((B, 1, nb, F), lambda j: (0, j, 0, 0))],
        out_specs=pl.BlockSpec((1, B, nb), lambda j: (j, 0, 0)),
        out_shape=jax.ShapeDtypeStruct((nj, B, nb), jnp.int32),
    )(x.reshape(B, nj, nb, F))
    return dd_t.transpose(1, 0, 2).reshape(B, N)


def _zero_hist(hist_ref):
    zeros = jnp.zeros((L,), jnp.int32)

    def zbody(i, _):
        hist_ref[pl.ds(i * L, L)] = zeros
        return 0

    lax.fori_loop(0, NBIN, zbody, 0)


def _threshold_scan(hist_ref, target, iota):
    def body(q, carry):
        acc, t_bin, cb, found = carry
        base = (q * L + iota) * L
        acc16 = jnp.zeros((L,), jnp.int32)
        for l in range(L):
            acc16 = acc16 + plsc.load_gather(hist_ref, [base + l])
        inc = plsc.cumsum(acc16)
        excl = inc - acc16
        crossv = (acc + inc) >= target
        nset = _s0(plsc.all_reduce_population_count(crossv))
        lane = _s0(plsc.all_reduce_ffs(crossv))
        hit = jnp.logical_and(found == 0, nset > 0)
        t_q = q * L + lane
        cb_q = acc + jnp.sum(jnp.where(iota == lane, excl, 0))
        t_bin = jnp.where(hit, t_q, t_bin)
        cb = jnp.where(hit, cb_q, cb)
        acc = acc + jnp.sum(acc16)
        found = jnp.where(nset > 0, 1, found)
        return acc, t_bin, cb, found

    _, t_bin, cb, _ = lax.fori_loop(
        0, NBIN // L, body, (jnp.int32(0), jnp.int32(0), jnp.int32(0),
                             jnp.int32(0)))
    return t_bin, cb


def _sc_body(dd_hbm, xr_hbm, out_hbm, dd_ref, hist_ref, off_ref,
             cand_d, cand_i, aux_d, aux_i, gi_ref, pgi_ref, pairs_ref,
             rows_ref, sem):
    c = lax.axis_index("c")
    s = lax.axis_index("s")
    active = s < 8
    b = c * 8 + s
    iota = lax.iota(jnp.int32, L)
    ones = jnp.ones((L,), jnp.int32)

    @pl.when(active)
    def _():
        pltpu.sync_copy(dd_hbm.at[pl.ds(b * N, N)], dd_ref)

        def level(lvl, prefix, target):
            shift = 24 - 8 * lvl
            _zero_hist(hist_ref)
            prefix_u = lax.convert_element_type(prefix, jnp.uint32)

            def hbody(v, _):
                dv = plsc.bitcast(dd_ref[pl.ds(v * L, L)], jnp.uint32)
                g = lax.shift_right_logical(dv, jnp.uint32(shift))
                g = g & jnp.uint32(0xFF)
                gidx = plsc.bitcast(g, jnp.int32) * L + iota
                if lvl == 0:
                    plsc.addupdate_scatter(hist_ref, [gidx], ones)
                else:
                    m = lax.shift_right_logical(
                        dv, jnp.uint32(shift + 8)) == prefix_u
                    plsc.addupdate_scatter(hist_ref, [gidx], ones, mask=m)
                return 0

            lax.fori_loop(0, NV, hbody, 0)
            t_bin, cb = _threshold_scan(hist_ref, target, iota)
            return t_bin, cb

        tgt1 = jnp.int32(K)
        t1, c1 = level(0, jnp.int32(0), tgt1)
        tgt2 = tgt1 - c1
        t2, c2 = level(1, t1, tgt2)
        tgt3 = tgt2 - c2
        p3 = t1 * 256 + t2
        t3, c3 = level(2, p3, tgt3)
        tgt4 = tgt3 - c3
        p4 = p3 * 256 + t3
        t4, c4 = level(3, p4, tgt4)
        s_cnt = c1 + c2 + c3 + c4
        dstar = lax.convert_element_type(p4, jnp.uint32) * jnp.uint32(256)
        dstar = dstar | lax.convert_element_type(t4, jnp.uint32)

        def cbody(v, carry):
            ps, pt = carry
            dvi = dd_ref[pl.ds(v * L, L)]
            dv = plsc.bitcast(dvi, jnp.uint32)
            iv = iota + v * L
            sel = dv < dstar
            plsc.store_compressed(cand_d.at[pl.ds(ps, L)], dvi, mask=sel)
            plsc.store_compressed(cand_i.at[pl.ds(ps, L)], iv, mask=sel)
            ps = ps + _s0(plsc.all_reduce_population_count(sel))
            tie = dv == dstar
            rank = pt + plsc.cumsum(ones, mask=tie) - 1
            tie = jnp.logical_and(tie, rank < K)
            plsc.store_compressed(cand_d.at[pl.ds(pt, L)], dvi, mask=tie)
            plsc.store_compressed(cand_i.at[pl.ds(pt, L)], iv, mask=tie)
            pt = pt + _s0(plsc.all_reduce_population_count(tie))
            return ps, pt

        lax.fori_loop(0, NV, cbody, (jnp.int32(0), s_cnt))

        lbase = iota * KL
        bufs = [(cand_d, cand_i), (aux_d, aux_i)]
        for p in range(4):
            src_d, src_i = bufs[p % 2]
            dst_d, dst_i = bufs[(p + 1) % 2]
            shift = jnp.uint32(8 * p)
            _zero_hist(hist_ref)

            def hbody(v, _, src_d=src_d, shift=shift):
                dv = plsc.bitcast(plsc.load_gather(src_d, [lbase + v]),
                                  jnp.uint32)
                g = lax.shift_right_logical(dv, shift) & jnp.uint32(0xFF)
                gl = plsc.bitcast(g, jnp.int32) * L + iota
                plsc.addupdate_scatter(hist_ref, [gl], ones)
                return 0

            lax.fori_loop(0, KL, hbody, 0)

            def obody(q, carry):
                h16 = hist_ref[pl.ds(q * L, L)]
                inc = plsc.cumsum(h16)
                off_ref[pl.ds(q * L, L)] = carry + inc - h16
                return carry + jnp.sum(h16)

            lax.fori_loop(0, NBIN, obody, jnp.int32(0))

            def pbody(v, _, src_d=src_d, src_i=src_i, dst_d=dst_d,
                      dst_i=dst_i, shift=shift):
                idxs = lbase + v
                dvi = plsc.load_gather(src_d, [idxs])
                ivi = plsc.load_gather(src_i, [idxs])
                g = lax.shift_right_logical(
                    plsc.bitcast(dvi, jnp.uint32), shift) & jnp.uint32(0xFF)
                gl = plsc.bitcast(g, jnp.int32) * L + iota
                cur = plsc.load_gather(off_ref, [gl])
                plsc.store_scatter(dst_d, [cur], dvi)
                plsc.store_scatter(dst_i, [cur], ivi)
                plsc.store_scatter(off_ref, [gl], cur + ones)
                return 0

            lax.fori_loop(0, KL, pbody, 0)

        base_row = b * N

        def gbody(v, _):
            gv = cand_i[pl.ds(v * L, L)] + base_row
            gi_ref[pl.ds(v * L, L)] = gv
            pgi_ref[pl.ds(v * L, L)] = lax.shift_right_logical(gv, 1)
            return 0

        lax.fori_loop(0, KL, gbody, 0)

        for ch in range(NCH):
            cp = pltpu.async_copy(
                xr_hbm.at[pgi_ref.at[pl.ds(ch * GCH, GCH)]], pairs_ref, sem)
            cp.wait()

            def hbody(t, _):
                f = t * L + iota
                j = lax.shift_right_logical(f, 6)
                u = f & (F - 1)
                par = plsc.load_gather(gi_ref, [ch * GCH + j]) & 1
                vec = plsc.load_gather(pairs_ref, [j, par * F + u])
                rows_ref[pl.ds(t * L, L)] = vec
                return 0

            lax.fori_loop(0, GCH * F // L, hbody, 0)
            pltpu.sync_copy(
                rows_ref,
                out_hbm.at[pl.ds((b * K + ch * GCH) * F, GCH * F)])


@functools.partial(jax.jit, static_argnums=())
def _sc_stage(dd, xr):
    mesh = plsc.VectorSubcoreMesh(core_axis_name="c", subcore_axis_name="s")
    kern = pl.kernel(
        _sc_body,
        out_type=jax.ShapeDtypeStruct((B * K * F,), jnp.float32),
        mesh=mesh,
        scratch_types=[
            pltpu.VMEM((N,), jnp.int32),
            pltpu.VMEM((NBIN * L,), jnp.int32),
            pltpu.VMEM((NBIN * L,), jnp.int32),
            pltpu.VMEM((CAND,), jnp.int32),
            pltpu.VMEM((CAND,), jnp.int32),
            pltpu.VMEM((CAND,), jnp.int32),
            pltpu.VMEM((CAND,), jnp.int32),
            pltpu.VMEM((K,), jnp.int32),
            pltpu.VMEM((K,), jnp.int32),
            pltpu.VMEM((GCH, 2 * F), jnp.float32),
            pltpu.VMEM((GCH * F,), jnp.float32),
            pltpu.SemaphoreType.DMA,
        ],
        compiler_params=pltpu.CompilerParams(needs_layout_passes=False),
    )
    return kern(dd, xr)


def kernel(x):
    dd = _keys_tc(x).reshape(B * N)
    xr = x.reshape(B * N // 2, 2 * F)
    out = _sc_stage(dd, xr)
    return out.reshape(B, K, F)
